# Initial kernel scaffold; baseline (speedup 1.0000x reference)
import jax
import jax.numpy as jnp
from jax.experimental import pallas as pl


def kernel(z, period_mapping, group_mapping):
    raise NotImplementedError("write your pallas kernel here")


# speedup vs baseline: 39.1895x; 39.1895x over previous
import jax
import jax.numpy as jnp
from jax import lax
from jax.experimental import pallas as pl
from jax.experimental.pallas import tpu as pltpu
from jax.experimental.pallas import tpu_sc as plsc

N_ATOMS = 100000
LANES = 16
NUM_WORKERS = 32
CHUNK = 3200
N_PAD = NUM_WORKERS * CHUNK
TAB_PAD = 96


def _tec_body(z_hbm, ptab_hbm, gtab_hbm, outp_hbm, outg_hbm,
              zv, ptab, gtab, outp, outg):
    wid = lax.axis_index("s") * 2 + lax.axis_index("c")
    base = wid * CHUNK
    pltpu.sync_copy(ptab_hbm, ptab)
    pltpu.sync_copy(gtab_hbm, gtab)
    pltpu.sync_copy(z_hbm.at[pl.ds(base, CHUNK)], zv)

    def step(i, carry):
        sl = pl.ds(i * LANES, LANES)
        idx = zv[sl]
        outp[sl] = plsc.load_gather(ptab, [idx])
        outg[sl] = plsc.load_gather(gtab, [idx])
        return carry

    lax.fori_loop(0, CHUNK // LANES, step, 0)

    pltpu.sync_copy(outp, outp_hbm.at[pl.ds(base, CHUNK)])
    pltpu.sync_copy(outg, outg_hbm.at[pl.ds(base, CHUNK)])


_mesh = plsc.VectorSubcoreMesh(
    core_axis_name="c", subcore_axis_name="s", num_cores=2, num_subcores=16)

_sc_call = pl.kernel(
    _tec_body,
    out_type=(
        jax.ShapeDtypeStruct((N_PAD,), jnp.int32),
        jax.ShapeDtypeStruct((N_PAD,), jnp.int32),
    ),
    mesh=_mesh,
    scratch_types=[
        pltpu.VMEM((CHUNK,), jnp.int32),
        pltpu.VMEM((TAB_PAD,), jnp.int32),
        pltpu.VMEM((TAB_PAD,), jnp.int32),
        pltpu.VMEM((CHUNK,), jnp.int32),
        pltpu.VMEM((CHUNK,), jnp.int32),
    ],
    compiler_params=pltpu.CompilerParams(needs_layout_passes=False),
)


def kernel(z, period_mapping, group_mapping):
    z_pad = jnp.concatenate(
        [z.astype(jnp.int32), jnp.zeros((N_PAD - N_ATOMS,), jnp.int32)])
    ptab = jnp.concatenate(
        [period_mapping,
         jnp.zeros((TAB_PAD - period_mapping.shape[0],), jnp.int32)])
    gtab = jnp.concatenate(
        [group_mapping,
         jnp.zeros((TAB_PAD - group_mapping.shape[0],), jnp.int32)])
    period_pad, group_pad = _sc_call(z_pad, ptab, gtab)
    return period_pad[:N_ATOMS], group_pad[:N_ATOMS]


# speedup vs baseline: 45.1805x; 1.1529x over previous
import jax
import jax.numpy as jnp
from jax import lax
from jax.experimental import pallas as pl
from jax.experimental.pallas import tpu as pltpu
from jax.experimental.pallas import tpu_sc as plsc

N_ATOMS = 100000
LANES = 16
NUM_WORKERS = 32
CHUNK = 3200
LAST_BASE = N_ATOMS - CHUNK
UNROLL = 4
N_TAB = 86


def _tec_body(z_hbm, ptab_hbm, gtab_hbm, outp_hbm, outg_hbm,
              zv, ptab, gtab, outp, outg):
    wid = lax.axis_index("s") * 2 + lax.axis_index("c")
    base = jnp.minimum(wid * CHUNK, LAST_BASE)
    pltpu.sync_copy(ptab_hbm, ptab)
    pltpu.sync_copy(gtab_hbm, gtab)
    pltpu.sync_copy(z_hbm.at[pl.ds(base, CHUNK)], zv)

    def step(i, carry):
        for u in range(UNROLL):
            sl = pl.ds((i * UNROLL + u) * LANES, LANES)
            idx = zv[sl]
            outp[sl] = plsc.load_gather(ptab, [idx])
            outg[sl] = plsc.load_gather(gtab, [idx])
        return carry

    lax.fori_loop(0, CHUNK // (LANES * UNROLL), step, 0)

    pltpu.sync_copy(outp, outp_hbm.at[pl.ds(base, CHUNK)])
    pltpu.sync_copy(outg, outg_hbm.at[pl.ds(base, CHUNK)])


_mesh = plsc.VectorSubcoreMesh(
    core_axis_name="c", subcore_axis_name="s", num_cores=2, num_subcores=16)

_sc_call = pl.kernel(
    _tec_body,
    out_type=(
        jax.ShapeDtypeStruct((N_ATOMS,), jnp.int32),
        jax.ShapeDtypeStruct((N_ATOMS,), jnp.int32),
    ),
    mesh=_mesh,
    scratch_types=[
        pltpu.VMEM((CHUNK,), jnp.int32),
        pltpu.VMEM((N_TAB,), jnp.int32),
        pltpu.VMEM((N_TAB,), jnp.int32),
        pltpu.VMEM((CHUNK,), jnp.int32),
        pltpu.VMEM((CHUNK,), jnp.int32),
    ],
    compiler_params=pltpu.CompilerParams(needs_layout_passes=False),
)


def kernel(z, period_mapping, group_mapping):
    return _sc_call(z, period_mapping, group_mapping)


# speedup vs baseline: 49.7516x; 1.1012x over previous
import jax
import jax.numpy as jnp
from jax import lax
from jax.experimental import pallas as pl
from jax.experimental.pallas import tpu as pltpu
from jax.experimental.pallas import tpu_sc as plsc

N_ATOMS = 100000
LANES = 16
NUM_WORKERS = 32
CHUNK = 3200
LAST_BASE = N_ATOMS - CHUNK
UNROLL = 4
N_TAB = 86


def _tec_body(z_hbm, ptab_hbm, gtab_hbm, outp_hbm, outg_hbm,
              zv, ptab, gtab, outp, outg, sem_in, sem_out):
    wid = lax.axis_index("s") * 2 + lax.axis_index("c")
    base = jnp.minimum(wid * CHUNK, LAST_BASE)
    cp_p = pltpu.async_copy(ptab_hbm, ptab, sem_in)
    cp_g = pltpu.async_copy(gtab_hbm, gtab, sem_in)
    cp_z = pltpu.async_copy(z_hbm.at[pl.ds(base, CHUNK)], zv, sem_in)
    cp_p.wait()
    cp_g.wait()
    cp_z.wait()

    @plsc.parallel_loop(0, CHUNK // LANES, unroll=UNROLL)
    def _step(i):
        sl = pl.ds(i * LANES, LANES)
        idx = zv[sl]
        outp[sl] = plsc.load_gather(ptab, [idx])
        outg[sl] = plsc.load_gather(gtab, [idx])

    cp_op = pltpu.async_copy(outp, outp_hbm.at[pl.ds(base, CHUNK)], sem_out)
    cp_og = pltpu.async_copy(outg, outg_hbm.at[pl.ds(base, CHUNK)], sem_out)
    cp_op.wait()
    cp_og.wait()


_mesh = plsc.VectorSubcoreMesh(
    core_axis_name="c", subcore_axis_name="s", num_cores=2, num_subcores=16)

_sc_call = pl.kernel(
    _tec_body,
    out_type=(
        jax.ShapeDtypeStruct((N_ATOMS,), jnp.int32),
        jax.ShapeDtypeStruct((N_ATOMS,), jnp.int32),
    ),
    mesh=_mesh,
    scratch_types=[
        pltpu.VMEM((CHUNK,), jnp.int32),
        pltpu.VMEM((N_TAB,), jnp.int32),
        pltpu.VMEM((N_TAB,), jnp.int32),
        pltpu.VMEM((CHUNK,), jnp.int32),
        pltpu.VMEM((CHUNK,), jnp.int32),
        pltpu.SemaphoreType.DMA,
        pltpu.SemaphoreType.DMA,
    ],
    compiler_params=pltpu.CompilerParams(needs_layout_passes=False),
)


def kernel(z, period_mapping, group_mapping):
    return _sc_call(z, period_mapping, group_mapping)
